# Initial kernel scaffold; baseline (speedup 1.0000x reference)
import jax
import jax.numpy as jnp
from jax.experimental import pallas as pl


def kernel(x, edge_index, W0, b0, W1, b1, W2, b2):
    raise NotImplementedError("write your pallas kernel here")


# speedup vs baseline: 10.7772x; 10.7772x over previous
import functools

import jax
import jax.numpy as jnp
from jax import lax
from jax.experimental import pallas as pl
from jax.experimental.pallas import tpu as pltpu
from jax.experimental.pallas import tpu_sc as plsc

N = 10000
E = 320000
D = 128
C = 2
S = 16
K = 80
NP = 10240
RT = NP // S
N2 = 10240
RT2 = N2 // S

_mesh = plsc.VectorSubcoreMesh(core_axis_name="c", subcore_axis_name="s")
_sc_params = pltpu.CompilerParams(use_tc_tiling_on_sc=False)
f32 = jnp.float32


@functools.partial(
    pl.kernel,
    out_type=jax.ShapeDtypeStruct((C * N2,), f32),
    mesh=_mesh,
    compiler_params=_sc_params,
    scratch_types=[
        pltpu.VMEM_SHARED((N2,), f32),
        pltpu.VMEM((K,), jnp.int32),
        pltpu.VMEM((K,), f32),
        pltpu.VMEM((RT2,), f32),
    ],
)
def _deg_kernel(col_hbm, out_hbm, acc_sh, idx_c, ones_v, stage_v):
    c = lax.axis_index("c")
    s = lax.axis_index("s")
    for q in range(RT2 // 16):
        stage_v[pl.ds(q * 16, 16)] = jnp.zeros((16,), f32)
    for q in range(K // 16):
        ones_v[pl.ds(q * 16, 16)] = jnp.ones((16,), f32)
    pltpu.sync_copy(stage_v, acc_sh.at[pl.ds(s * RT2, RT2)])
    plsc.subcore_barrier()
    e_per_tile = E // (C * S)
    e_base = (s * C + c) * e_per_tile
    def step(i, carry):
        pltpu.sync_copy(col_hbm.at[pl.ds(e_base + i * K, K)], idx_c)
        pltpu.sync_copy(ones_v, acc_sh.at[idx_c], add=True)
        return carry
    lax.fori_loop(0, e_per_tile // K, step, 0)
    plsc.subcore_barrier()
    pltpu.sync_copy(acc_sh.at[pl.ds(s * RT2, RT2)], stage_v)
    pltpu.sync_copy(stage_v, out_hbm.at[pl.ds(c * N2 + s * RT2, RT2)])


@functools.partial(
    pl.kernel,
    out_type=jax.ShapeDtypeStruct((C * NP, D), f32),
    mesh=_mesh,
    compiler_params=_sc_params,
    scratch_types=[
        pltpu.VMEM_SHARED((NP, D), f32),
        pltpu.VMEM((K,), jnp.int32),
        pltpu.VMEM((K,), jnp.int32),
        pltpu.VMEM((K, D), f32),
        pltpu.SemaphoreType.DMA,
    ],
)
def _spmm_pair_kernel(g_hbm, row0_hbm, row1_hbm, col_hbm, out_hbm,
                      acc_sh, idx_r, idx_c, rows_v, sem):
    c = lax.axis_index("c")
    s = lax.axis_index("s")
    for j in range(RT // K):
        pltpu.sync_copy(g_hbm.at[pl.ds(c * NP + s * RT + j * K, K), :], rows_v)
        pltpu.sync_copy(rows_v, acc_sh.at[pl.ds(s * RT + j * K, K), :])
    plsc.subcore_barrier()
    e_per_tile = E // S
    e_base = s * e_per_tile
    def step(i, carry):
        base = e_base + i * K
        pltpu.sync_copy(col_hbm.at[pl.ds(base, K)], idx_c)
        @pl.when(c == 0)
        def _():
            pltpu.sync_copy(row0_hbm.at[pl.ds(base, K)], idx_r)
        @pl.when(c == 1)
        def _():
            pltpu.sync_copy(row1_hbm.at[pl.ds(base, K)], idx_r)
        pltpu.async_copy(g_hbm.at[idx_r], rows_v, sem).wait()
        pltpu.sync_copy(rows_v, acc_sh.at[idx_c], add=True)
        return carry
    lax.fori_loop(0, e_per_tile // K, step, 0)
    plsc.subcore_barrier()
    for j in range(RT // K):
        pltpu.sync_copy(acc_sh.at[pl.ds(s * RT + j * K, K), :], rows_v)
        pltpu.sync_copy(rows_v, out_hbm.at[pl.ds(c * NP + s * RT + j * K, K), :])


@functools.partial(
    pl.kernel,
    out_type=jax.ShapeDtypeStruct((C * NP, D), f32),
    mesh=_mesh,
    compiler_params=_sc_params,
    scratch_types=[
        pltpu.VMEM_SHARED((NP, D), f32),
        pltpu.VMEM((K,), jnp.int32),
        pltpu.VMEM((K,), jnp.int32),
        pltpu.VMEM((K, D), f32),
        pltpu.SemaphoreType.DMA,
    ],
)
def _spmm_split_kernel(v_hbm, row_hbm, col_hbm, out_hbm,
                       acc_sh, idx_r, idx_c, rows_v, sem):
    c = lax.axis_index("c")
    s = lax.axis_index("s")
    for j in range(RT // K):
        pltpu.sync_copy(v_hbm.at[pl.ds(s * RT + j * K, K), :], rows_v)
        pltpu.sync_copy(rows_v, acc_sh.at[pl.ds(s * RT + j * K, K), :])
    plsc.subcore_barrier()
    e_per_tile = E // (C * S)
    e_base = (s * C + c) * e_per_tile
    def step(i, carry):
        base = e_base + i * K
        pltpu.sync_copy(col_hbm.at[pl.ds(base, K)], idx_c)
        pltpu.sync_copy(row_hbm.at[pl.ds(base, K)], idx_r)
        pltpu.async_copy(v_hbm.at[idx_r], rows_v, sem).wait()
        pltpu.sync_copy(rows_v, acc_sh.at[idx_c], add=True)
        return carry
    lax.fori_loop(0, e_per_tile // K, step, 0)
    plsc.subcore_barrier()
    for j in range(RT // K):
        pltpu.sync_copy(acc_sh.at[pl.ds(s * RT + j * K, K), :], rows_v)
        pltpu.sync_copy(rows_v, out_hbm.at[pl.ds(c * NP + s * RT + j * K, K), :])


_B = 2000


def _s_body(degp_ref, s_ref):
    deg = 1.0 + degp_ref[0, :] + degp_ref[1, :]
    s_ref[...] = lax.rsqrt(deg)[:, None]


def _tc_s(degp):
    b2 = 2048
    return pl.pallas_call(
        _s_body,
        grid=(N2 // b2,),
        in_specs=[pl.BlockSpec((2, b2), lambda i: (0, i))],
        out_specs=pl.BlockSpec((b2, 1), lambda i: (i, 0)),
        out_shape=jax.ShapeDtypeStruct((N2, 1), f32),
    )(degp)


def _fused_body(x_ref, wc_ref, bc_ref, s_ref, h0_ref, g_ref):
    h = jnp.dot(x_ref[...], wc_ref[...], preferred_element_type=f32)
    h = h + bc_ref[...]
    sc = s_ref[...]
    h0_ref[...] = h[:, :D]
    g_ref[0] = sc * h[:, D:2 * D]
    g_ref[1] = sc * h[:, 2 * D:]


def _mid_body(m_ref, s_ref, t1_ref, v_ref):
    sc = s_ref[...]
    t1_ref[...] = sc * m_ref[0]
    v_ref[...] = (sc * sc) * m_ref[1]


def _final_body(m_ref, v_ref, s_ref, y2_ref):
    y2_ref[...] = s_ref[...] * (m_ref[0] + m_ref[1] - v_ref[...])


def _tc_fused(x, wc, bc, s):
    return pl.pallas_call(
        _fused_body,
        grid=(N // _B,),
        in_specs=[
            pl.BlockSpec((_B, D), lambda i: (i, 0)),
            pl.BlockSpec((D, 3 * D), lambda i: (0, 0)),
            pl.BlockSpec((1, 3 * D), lambda i: (0, 0)),
            pl.BlockSpec((_B, 1), lambda i: (i, 0)),
        ],
        out_specs=[
            pl.BlockSpec((_B, D), lambda i: (i, 0)),
            pl.BlockSpec((2, _B, D), lambda i: (0, i, 0)),
        ],
        out_shape=[
            jax.ShapeDtypeStruct((N, D), f32),
            jax.ShapeDtypeStruct((2, NP, D), f32),
        ],
    )(x, wc, bc, s)


def _tc_mid(m, s):
    return pl.pallas_call(
        _mid_body,
        grid=(N // _B,),
        in_specs=[
            pl.BlockSpec((2, _B, D), lambda i: (0, i, 0)),
            pl.BlockSpec((_B, 1), lambda i: (i, 0)),
        ],
        out_specs=[
            pl.BlockSpec((_B, D), lambda i: (i, 0)),
            pl.BlockSpec((_B, D), lambda i: (i, 0)),
        ],
        out_shape=[
            jax.ShapeDtypeStruct((N, D), f32),
            jax.ShapeDtypeStruct((NP, D), f32),
        ],
    )(m, s)


def _tc_final(m, v, s):
    return pl.pallas_call(
        _final_body,
        grid=(N // _B,),
        in_specs=[
            pl.BlockSpec((2, _B, D), lambda i: (0, i, 0)),
            pl.BlockSpec((_B, D), lambda i: (i, 0)),
            pl.BlockSpec((_B, 1), lambda i: (i, 0)),
        ],
        out_specs=pl.BlockSpec((_B, D), lambda i: (i, 0)),
        out_shape=jax.ShapeDtypeStruct((N, D), f32),
    )(m, v, s)


def kernel(x, edge_index, W0, b0, W1, b1, W2, b2):
    row = edge_index[0]
    col = edge_index[1]
    row1 = row + NP
    wc = jnp.concatenate([W0, W1, W2], axis=0).T
    bc = jnp.concatenate([b0, b1, b2]).reshape(1, 3 * D)

    degp = _deg_kernel(col).reshape(C, N2)
    s = _tc_s(degp)
    h0, g = _tc_fused(x, wc, bc, s)

    m1 = _spmm_pair_kernel(g.reshape(C * NP, D), row, row1, col)
    t1, v = _tc_mid(m1.reshape(C, NP, D), s)

    m2 = _spmm_split_kernel(v, row, col)
    y2 = _tc_final(m2.reshape(C, NP, D), v, s)

    return jnp.concatenate([h0, t1, y2], axis=1)
